# baseline (speedup 1.0000x reference)
import functools

import jax
import jax.numpy as jnp
from jax import lax
from jax.experimental import pallas as pl
from jax.experimental.pallas import tpu as pltpu
from jax.experimental.pallas import tpu_sc as plsc

BATCH = 16384
TASKS = 16
NC = 2
NS = 16
NW = NC * NS
BPW = BATCH // NW
CHUNK = 128
NCHUNK = BPW // CHUNK


def _softplus16(w):
    u = jnp.exp(-jnp.abs(w))
    z = u / (u + 2.0)
    z2 = z * z
    poly = 1.0 + z2 * (1.0 / 3.0 + z2 * (1.0 / 5.0 + z2 * (1.0 / 7.0 + z2 * (1.0 / 9.0))))
    return jnp.maximum(w, 0.0) + 2.0 * z * poly


def _sc_body(loss_hbm, idx_hbm, table_hbm, out_hbm,
             idx_v, loss_v, rows_v, acc_v, gsem, lsem):
    wid = lax.axis_index("s") * NC + lax.axis_index("c")
    pltpu.sync_copy(idx_hbm.at[wid], idx_v)
    lcp = pltpu.async_copy(loss_hbm.at[wid], loss_v, lsem)
    gcps = [
        pltpu.async_copy(table_hbm.at[idx_v.at[j]],
                         rows_v.at[pl.ds(j * CHUNK, CHUNK)], gsem)
        for j in range(NCHUNK)
    ]
    for cp in gcps:
        cp.wait()
    lcp.wait()

    def body(k, acc):
        r = k * 8
        terms = []
        for j in range(8):
            w = rows_v[r + j, :]
            l = loss_v[r + j, :]
            terms.append(_softplus16(w) * l)
        t01 = terms[0] + terms[1]
        t23 = terms[2] + terms[3]
        t45 = terms[4] + terms[5]
        t67 = terms[6] + terms[7]
        return acc + ((t01 + t23) + (t45 + t67))

    acc = lax.fori_loop(0, BPW // 8, body, jnp.zeros((TASKS,), jnp.float32))
    acc_v[...] = acc * (1.0 / BATCH)
    pltpu.sync_copy(acc_v, out_hbm.at[wid])


_sc_partials = functools.partial(
    pl.kernel,
    out_type=jax.ShapeDtypeStruct((NW, TASKS), jnp.float32),
    mesh=plsc.VectorSubcoreMesh(core_axis_name="c", subcore_axis_name="s"),
    compiler_params=pltpu.CompilerParams(use_tc_tiling_on_sc=False),
    scratch_types=[
        pltpu.VMEM((NCHUNK, CHUNK), jnp.int32),
        pltpu.VMEM((BPW, TASKS), jnp.float32),
        pltpu.VMEM((BPW, TASKS), jnp.float32),
        pltpu.VMEM((TASKS,), jnp.float32),
        pltpu.SemaphoreType.DMA,
        pltpu.SemaphoreType.DMA,
    ],
)(_sc_body)


def _tc_sum_body(x_ref, o_ref):
    o_ref[0, 0] = jnp.sum(x_ref[...])


_tc_sum = pl.pallas_call(
    _tc_sum_body,
    out_shape=jax.ShapeDtypeStruct((1, 1), jnp.float32),
    out_specs=pl.BlockSpec(memory_space=pltpu.SMEM),
)


def kernel(losses, sample_id, weights_table):
    idx = jnp.reshape(sample_id.astype(jnp.int32), (NW, NCHUNK, CHUNK))
    loss_r = jnp.reshape(losses, (NW, BPW, TASKS))
    partials = _sc_partials(loss_r, idx, weights_table)
    total = _tc_sum(partials)
    return total[0, 0]
